# Initial kernel scaffold; baseline (speedup 1.0000x reference)
import jax
import jax.numpy as jnp
from jax.experimental import pallas as pl


def kernel(box, score, feat, Wm, bm, Wmo, bmo, Wp, bp, Wpo, bpo, Wg, bg, Wgo, bgo):
    raise NotImplementedError("write your pallas kernel here")


# speedup vs baseline: 1.0390x; 1.0390x over previous
import jax
import jax.numpy as jnp
from jax.experimental import pallas as pl
from jax.experimental.pallas import tpu as pltpu

_BIN = 1000
_RB = 8
_R2 = 2
_THRS = (0.4, 0.6, 0.8)
_GAMMA_F = 0.05
_RESCALE = 0.02
_EPS = 1e-12
_BIG = float(1 << 20)
_PAD = float(1 << 24)


def _lrelu(x):
    return jnp.where(x >= 0, x, 0.2 * x)


def _dot(a, b):
    return jax.lax.dot_general(
        a, b, (((1,), (0,)), ((), ())),
        precision=jax.lax.Precision.HIGHEST,
        preferred_element_type=jnp.float32)


def _iou(x1a, y1a, x2a, y2a, aa, x1b, y1b, x2b, y2b, ab):
    wx = jnp.maximum(jnp.minimum(x2a, x2b) - jnp.maximum(x1a, x1b), 0.0)
    wy = jnp.maximum(jnp.minimum(y2a, y2b) - jnp.maximum(y1a, y1b), 0.0)
    inter = wx * wy
    return inter / ((aa + ab) - inter)


def _kernel(nrows, x1c_ref, y1c_ref, x2c_ref, y2c_ref, ac_ref, fl_ref,
            x1t_ref, y1t_ref, x2t_ref, y2t_ref, at_ref, ut_ref, bt_ref,
            feat_ref, Wm_ref, bm_ref, Wmo_ref, bmo_ref,
            Wp_ref, bp_ref, Wpo_ref, bpo_ref,
            Wg_ref, bg_ref, Wgo_ref, bgo_ref,
            pi_ref, mu_ref, gamma_ref, loc_ref):
    x1c = x1c_ref[0]
    y1c = y1c_ref[0]
    x2c = x2c_ref[0]
    y2c = y2c_ref[0]
    ac = ac_ref[0]

    iou_rg = _iou(fl_ref[0], fl_ref[1], fl_ref[2], fl_ref[3], fl_ref[4],
                  x1c, y1c, x2c, y2c, ac)
    K = [jnp.where(iou_rg >= t, fl_ref[5], fl_ref[6]) for t in _THRS]

    row_refs = (x1t_ref, y1t_ref, x2t_ref, y2t_ref, at_ref, ut_ref, bt_ref)

    def rowstep(k, bad):
        lane = k * _R2
        base = pl.multiple_of((lane // 128) * 128, 128)
        off = lane - base
        tiles = [pltpu.roll(r[:, pl.ds(base, 128)], -off, axis=1)
                 for r in row_refs]
        out = bad
        for j in range(_R2):
            sj = slice(j, j + 1)
            x1i, y1i, x2i, y2i, ai, ui, bi = [t[:, sj] for t in tiles]
            iou = _iou(x1i, y1i, x2i, y2i, ai, x1c, y1c, x2c, y2c, ac)
            nxt = []
            for t, b, k_t in zip(_THRS, out, K):
                key = jnp.where(iou >= t, ui, bi)
                nxt.append(jnp.where(key < k_t, 1.0, b))
            out = tuple(nxt)
        return out

    nrowsteps = nrows // (_RB * _R2)
    z = jnp.zeros((_RB, _BIN), dtype=jnp.float32)
    bad = jax.lax.fori_loop(0, nrowsteps, rowstep, (z, z, z))
    masks = [1.0 - jnp.max(b, axis=0, keepdims=True) for b in bad]

    feat = feat_ref[0]

    def stack(h, Ws_ref, bs_ref, Wo_ref, bo_ref):
        for i in range(Ws_ref.shape[0]):
            h = _lrelu(_dot(Ws_ref[i], h) + bs_ref[i])
        return _dot(Wo_ref[...], h) + bo_ref[...]

    om = stack(feat, Wm_ref, bm_ref, Wmo_ref, bmo_ref)
    mx = jnp.max(om, axis=0, keepdims=True)
    e = jnp.exp(om - mx)
    w = e / jnp.sum(e, axis=0, keepdims=True)
    loc_ref[0] = (masks[0] * w[0:1] + masks[1] * w[1:2] + masks[2] * w[2:3])

    op = stack(feat, Wp_ref, bp_ref, Wpo_ref, bpo_ref)
    pi_ref[0] = jnp.exp(op)

    og = stack(feat, Wg_ref, bg_ref, Wgo_ref, bgo_ref)
    sp = jnp.maximum(og, 0.0) + jnp.log1p(jnp.exp(-jnp.abs(og)))
    mu1 = _RESCALE * x1c
    mu2 = _RESCALE * y1c
    mu3 = _RESCALE * x2c
    mu4 = _RESCALE * y2c
    dx = jnp.maximum(mu3 - mu1, _EPS)
    dy = jnp.maximum(mu4 - mu2, _EPS)
    ming = _GAMMA_F * jnp.concatenate([dx, dy, dx, dy], axis=0)
    gamma_ref[0] = sp + ming
    mu_ref[0] = jnp.concatenate([mu1, mu2, mu3, mu4], axis=0)


def kernel(box, score, feat, Wm, bm, Wmo, bmo, Wp, bp, Wpo, bpo, Wg, bg, Wgo, bgo):
    n = box.shape[2]
    c = feat.shape[1]
    nblk = n // _BIN
    ncol = n // _RB
    ncol_pad = -(-ncol // 128) * 128
    npad = ncol_pad * _RB

    x1 = box[:, 0]
    y1 = box[:, 1]
    x2 = box[:, 2]
    y2 = box[:, 3]
    area = (x2 - x1) * (y2 - y1)

    s = score[0, 0]
    idx = jnp.arange(n, dtype=jnp.int32)
    order = jnp.argsort(-s, stable=True)
    srank = jnp.zeros((n,), jnp.int32).at[order].set(idx)
    u = jnp.where(s > 0, srank.astype(jnp.float32),
                  _BIG + idx.astype(jnp.float32))[None, :]
    bigi = (_BIG + idx.astype(jnp.float32))[None, :]

    def rowt(a, val):
        ap = jnp.pad(a, ((0, 0), (0, npad - n)), constant_values=val)
        return ap.reshape(ncol_pad, _RB).T

    row_args = [rowt(x1, 0.), rowt(y1, 0.), rowt(x2, 0.), rowt(y2, 0.),
                rowt(area, 0.), rowt(u, _PAD), rowt(bigi, _PAD)]

    ref_l = jnp.stack([a[:, :_BIN]
                       for a in (x1, y1, x2, y2, area, u, bigi)])

    col3 = [a.reshape(nblk, 1, _BIN) for a in (x1, y1, x2, y2, area)]
    featb = feat[0].reshape(c, nblk, _BIN).transpose(1, 0, 2)

    col_spec = pl.BlockSpec((1, 1, _BIN), lambda j: (j, 0, 0))
    full2 = lambda a: pl.BlockSpec(a.shape, lambda j: (0,) * a.ndim)

    w_arrays = [Wm, bm[..., None], Wmo, bmo[:, None],
                Wp, bp[..., None], Wpo, bpo[:, None],
                Wg, bg[..., None], Wgo, bgo[:, None]]
    in_arrays = (col3 + [ref_l] + row_args + [featb] + w_arrays)
    in_specs = ([col_spec] * 5
                + [full2(ref_l)]
                + [full2(a) for a in row_args]
                + [pl.BlockSpec((1, c, _BIN), lambda j: (j, 0, 0))]
                + [full2(a) for a in w_arrays])

    out_shape = (
        jax.ShapeDtypeStruct((nblk, 1, _BIN), jnp.float32),
        jax.ShapeDtypeStruct((nblk, 4, _BIN), jnp.float32),
        jax.ShapeDtypeStruct((nblk, 4, _BIN), jnp.float32),
        jax.ShapeDtypeStruct((nblk, 1, _BIN), jnp.float32),
    )
    blk_out = lambda r: pl.BlockSpec((1, r, _BIN), lambda j: (j, 0, 0))
    out_specs = (blk_out(1), blk_out(4), blk_out(4), blk_out(1))

    import functools
    pi, mu, gamma, loc = pl.pallas_call(
        functools.partial(_kernel, n),
        grid=(nblk,),
        in_specs=in_specs,
        out_specs=out_specs,
        out_shape=out_shape,
        compiler_params=pltpu.CompilerParams(
            dimension_semantics=("arbitrary",)),
    )(*in_arrays)

    pi = pi.reshape(1, 1, n)
    mu = mu.transpose(1, 0, 2).reshape(1, 4, n)
    gamma = gamma.transpose(1, 0, 2).reshape(1, 4, n)
    loc = loc.reshape(1, 1, n)
    return (pi, mu, gamma, loc)


# speedup vs baseline: 2.9455x; 2.8351x over previous
import jax
import jax.numpy as jnp
from jax.experimental import pallas as pl
from jax.experimental.pallas import tpu as pltpu

_BIN = 1000
_RB = 8
_R2 = 2
_THRS = (0.4, 0.6, 0.8)
_GAMMA_F = 0.05
_RESCALE = 0.02
_EPS = 1e-12
_BIG = float(1 << 20)
_PAD = float(1 << 24)


def _lrelu(x):
    return jnp.where(x >= 0, x, 0.2 * x)


def _dot(a, b):
    return jax.lax.dot_general(
        a, b, (((1,), (0,)), ((), ())),
        precision=jax.lax.Precision.HIGHEST,
        preferred_element_type=jnp.float32)


def _iou(x1a, y1a, x2a, y2a, aa, x1b, y1b, x2b, y2b, ab):
    wx = jnp.maximum(jnp.minimum(x2a, x2b) - jnp.maximum(x1a, x1b), 0.0)
    wy = jnp.maximum(jnp.minimum(y2a, y2b) - jnp.maximum(y1a, y1b), 0.0)
    inter = wx * wy
    return inter / ((aa + ab) - inter)


def _kernel(nrows, x1c_ref, y1c_ref, x2c_ref, y2c_ref, ac_ref, fl_ref,
            x1t_ref, y1t_ref, x2t_ref, y2t_ref, at_ref, ut_ref, bt_ref,
            feat_ref, Wm_ref, bm_ref, Wmo_ref, bmo_ref,
            Wp_ref, bp_ref, Wpo_ref, bpo_ref,
            Wg_ref, bg_ref, Wgo_ref, bgo_ref,
            pi_ref, mu_ref, gamma_ref, loc_ref):
    x1c = x1c_ref[0]
    y1c = y1c_ref[0]
    x2c = x2c_ref[0]
    y2c = y2c_ref[0]
    ac = ac_ref[0]

    iou_rg = _iou(fl_ref[0], fl_ref[1], fl_ref[2], fl_ref[3], fl_ref[4],
                  x1c, y1c, x2c, y2c, ac)
    K = [jnp.where(iou_rg >= t, fl_ref[5], fl_ref[6]) for t in _THRS]

    row_refs = (x1t_ref, y1t_ref, x2t_ref, y2t_ref, at_ref, ut_ref, bt_ref)

    def rowstep(k, bad):
        base = pl.multiple_of(k * 128, 128)
        tiles = [r[:, pl.ds(base, 128)] for r in row_refs]
        out = bad
        for j in range(128):
            sj = slice(j, j + 1)
            x1i, y1i, x2i, y2i, ai, ui, bi = [t[:, sj] for t in tiles]
            iou = _iou(x1i, y1i, x2i, y2i, ai, x1c, y1c, x2c, y2c, ac)
            nxt = []
            for t, b, k_t in zip(_THRS, out, K):
                key = jnp.where(iou >= t, ui, bi)
                nxt.append(jnp.where(key < k_t, 1.0, b))
            out = tuple(nxt)
        return out

    nrowsteps = nrows
    z = jnp.zeros((_RB, _BIN), dtype=jnp.float32)
    bad = jax.lax.fori_loop(0, nrowsteps, rowstep, (z, z, z))
    masks = [1.0 - jnp.max(b, axis=0, keepdims=True) for b in bad]

    feat = feat_ref[0]

    def stack(h, Ws_ref, bs_ref, Wo_ref, bo_ref):
        for i in range(Ws_ref.shape[0]):
            h = _lrelu(_dot(Ws_ref[i], h) + bs_ref[i])
        return _dot(Wo_ref[...], h) + bo_ref[...]

    om = stack(feat, Wm_ref, bm_ref, Wmo_ref, bmo_ref)
    mx = jnp.max(om, axis=0, keepdims=True)
    e = jnp.exp(om - mx)
    w = e / jnp.sum(e, axis=0, keepdims=True)
    loc_ref[0] = (masks[0] * w[0:1] + masks[1] * w[1:2] + masks[2] * w[2:3])

    op = stack(feat, Wp_ref, bp_ref, Wpo_ref, bpo_ref)
    pi_ref[0] = jnp.exp(op)

    og = stack(feat, Wg_ref, bg_ref, Wgo_ref, bgo_ref)
    sp = jnp.maximum(og, 0.0) + jnp.log1p(jnp.exp(-jnp.abs(og)))
    mu1 = _RESCALE * x1c
    mu2 = _RESCALE * y1c
    mu3 = _RESCALE * x2c
    mu4 = _RESCALE * y2c
    dx = jnp.maximum(mu3 - mu1, _EPS)
    dy = jnp.maximum(mu4 - mu2, _EPS)
    ming = _GAMMA_F * jnp.concatenate([dx, dy, dx, dy], axis=0)
    gamma_ref[0] = sp + ming
    mu_ref[0] = jnp.concatenate([mu1, mu2, mu3, mu4], axis=0)


def kernel(box, score, feat, Wm, bm, Wmo, bmo, Wp, bp, Wpo, bpo, Wg, bg, Wgo, bgo):
    n = box.shape[2]
    c = feat.shape[1]
    nblk = n // _BIN
    ncol = n // _RB
    ncol_pad = -(-ncol // 128) * 128
    npad = ncol_pad * _RB

    x1 = box[:, 0]
    y1 = box[:, 1]
    x2 = box[:, 2]
    y2 = box[:, 3]
    area = (x2 - x1) * (y2 - y1)

    s = score[0, 0]
    idx = jnp.arange(n, dtype=jnp.int32)
    order = jnp.argsort(-s, stable=True)
    srank = jnp.zeros((n,), jnp.int32).at[order].set(idx)
    u = jnp.where(s > 0, srank.astype(jnp.float32),
                  _BIG + idx.astype(jnp.float32))[None, :]
    bigi = (_BIG + idx.astype(jnp.float32))[None, :]

    def rowt(a, val):
        ap = jnp.pad(a, ((0, 0), (0, npad - n)), constant_values=val)
        return ap.reshape(ncol_pad, _RB).T

    row_args = [rowt(x1, 0.), rowt(y1, 0.), rowt(x2, 0.), rowt(y2, 0.),
                rowt(area, 0.), rowt(u, _PAD), rowt(bigi, _PAD)]

    ref_l = jnp.stack([a[:, :_BIN]
                       for a in (x1, y1, x2, y2, area, u, bigi)])

    col3 = [a.reshape(nblk, 1, _BIN) for a in (x1, y1, x2, y2, area)]
    featb = feat[0].reshape(c, nblk, _BIN).transpose(1, 0, 2)

    col_spec = pl.BlockSpec((1, 1, _BIN), lambda j: (j, 0, 0))
    full2 = lambda a: pl.BlockSpec(a.shape, lambda j: (0,) * a.ndim)

    w_arrays = [Wm, bm[..., None], Wmo, bmo[:, None],
                Wp, bp[..., None], Wpo, bpo[:, None],
                Wg, bg[..., None], Wgo, bgo[:, None]]
    in_arrays = (col3 + [ref_l] + row_args + [featb] + w_arrays)
    in_specs = ([col_spec] * 5
                + [full2(ref_l)]
                + [full2(a) for a in row_args]
                + [pl.BlockSpec((1, c, _BIN), lambda j: (j, 0, 0))]
                + [full2(a) for a in w_arrays])

    out_shape = (
        jax.ShapeDtypeStruct((nblk, 1, _BIN), jnp.float32),
        jax.ShapeDtypeStruct((nblk, 4, _BIN), jnp.float32),
        jax.ShapeDtypeStruct((nblk, 4, _BIN), jnp.float32),
        jax.ShapeDtypeStruct((nblk, 1, _BIN), jnp.float32),
    )
    blk_out = lambda r: pl.BlockSpec((1, r, _BIN), lambda j: (j, 0, 0))
    out_specs = (blk_out(1), blk_out(4), blk_out(4), blk_out(1))

    import functools
    pi, mu, gamma, loc = pl.pallas_call(
        functools.partial(_kernel, ncol_pad // 128),
        grid=(nblk,),
        in_specs=in_specs,
        out_specs=out_specs,
        out_shape=out_shape,
        compiler_params=pltpu.CompilerParams(
            dimension_semantics=("arbitrary",)),
    )(*in_arrays)

    pi = pi.reshape(1, 1, n)
    mu = mu.transpose(1, 0, 2).reshape(1, 4, n)
    gamma = gamma.transpose(1, 0, 2).reshape(1, 4, n)
    loc = loc.reshape(1, 1, n)
    return (pi, mu, gamma, loc)


# speedup vs baseline: 3.2277x; 1.0958x over previous
import jax
import jax.numpy as jnp
from jax.experimental import pallas as pl
from jax.experimental.pallas import tpu as pltpu

_BIN = 1000
_RB = 8
_R2 = 2
_THRS = (0.4, 0.6, 0.8)
_GAMMA_F = 0.05
_RESCALE = 0.02
_EPS = 1e-12
_BIG = float(1 << 20)
_PAD = float(1 << 24)


def _lrelu(x):
    return jnp.where(x >= 0, x, 0.2 * x)


def _dot(a, b):
    return jax.lax.dot_general(
        a, b, (((1,), (0,)), ((), ())),
        precision=jax.lax.Precision.HIGHEST,
        preferred_element_type=jnp.float32)


def _iou(x1a, y1a, x2a, y2a, aa, x1b, y1b, x2b, y2b, ab):
    wx = jnp.maximum(jnp.minimum(x2a, x2b) - jnp.maximum(x1a, x1b), 0.0)
    wy = jnp.maximum(jnp.minimum(y2a, y2b) - jnp.maximum(y1a, y1b), 0.0)
    inter = wx * wy
    return inter / ((aa + ab) - inter)


def _kernel(nrows, x1c_ref, y1c_ref, x2c_ref, y2c_ref, ac_ref, fl_ref,
            x1t_ref, y1t_ref, x2t_ref, y2t_ref, at_ref, ut_ref, bt_ref,
            feat_ref, Wm_ref, bm_ref, Wmo_ref, bmo_ref,
            Wp_ref, bp_ref, Wpo_ref, bpo_ref,
            Wg_ref, bg_ref, Wgo_ref, bgo_ref,
            pi_ref, mu_ref, gamma_ref, loc_ref):
    x1c = x1c_ref[0]
    y1c = y1c_ref[0]
    x2c = x2c_ref[0]
    y2c = y2c_ref[0]
    ac = ac_ref[0]

    iou_rg = _iou(fl_ref[0], fl_ref[1], fl_ref[2], fl_ref[3], fl_ref[4],
                  x1c, y1c, x2c, y2c, ac)
    K = [jnp.where(iou_rg >= t, fl_ref[5], fl_ref[6]) for t in _THRS]

    row_refs = (x1t_ref, y1t_ref, x2t_ref, y2t_ref, at_ref, ut_ref, bt_ref)

    def rowstep(k, bad):
        base = pl.multiple_of(k * 128, 128)
        tiles = [r[:, pl.ds(base, 128)] for r in row_refs]
        out = bad
        for j in range(128):
            sj = slice(j, j + 1)
            x1i, y1i, x2i, y2i, ai, ui, bi = [t[:, sj] for t in tiles]
            iou = _iou(x1i, y1i, x2i, y2i, ai, x1c, y1c, x2c, y2c, ac)
            nxt = []
            for t, b in zip(_THRS, out):
                key = jnp.where(iou >= t, ui, bi)
                nxt.append(jnp.minimum(b, key))
            out = tuple(nxt)
        return out

    nrowsteps = nrows
    z = jnp.full((_RB, _BIN), _PAD, dtype=jnp.float32)
    minkey = jax.lax.fori_loop(0, nrowsteps, rowstep, (z, z, z))
    masks = [jnp.where(jnp.min(mk, axis=0, keepdims=True) < k_t, 0.0, 1.0)
             for mk, k_t in zip(minkey, K)]

    feat = feat_ref[0]

    def stack(h, Ws_ref, bs_ref, Wo_ref, bo_ref):
        for i in range(Ws_ref.shape[0]):
            h = _lrelu(_dot(Ws_ref[i], h) + bs_ref[i])
        return _dot(Wo_ref[...], h) + bo_ref[...]

    om = stack(feat, Wm_ref, bm_ref, Wmo_ref, bmo_ref)
    mx = jnp.max(om, axis=0, keepdims=True)
    e = jnp.exp(om - mx)
    w = e / jnp.sum(e, axis=0, keepdims=True)
    loc_ref[0] = (masks[0] * w[0:1] + masks[1] * w[1:2] + masks[2] * w[2:3])

    op = stack(feat, Wp_ref, bp_ref, Wpo_ref, bpo_ref)
    pi_ref[0] = jnp.exp(op)

    og = stack(feat, Wg_ref, bg_ref, Wgo_ref, bgo_ref)
    sp = jnp.maximum(og, 0.0) + jnp.log1p(jnp.exp(-jnp.abs(og)))
    mu1 = _RESCALE * x1c
    mu2 = _RESCALE * y1c
    mu3 = _RESCALE * x2c
    mu4 = _RESCALE * y2c
    dx = jnp.maximum(mu3 - mu1, _EPS)
    dy = jnp.maximum(mu4 - mu2, _EPS)
    ming = _GAMMA_F * jnp.concatenate([dx, dy, dx, dy], axis=0)
    gamma_ref[0] = sp + ming
    mu_ref[0] = jnp.concatenate([mu1, mu2, mu3, mu4], axis=0)


def kernel(box, score, feat, Wm, bm, Wmo, bmo, Wp, bp, Wpo, bpo, Wg, bg, Wgo, bgo):
    n = box.shape[2]
    c = feat.shape[1]
    nblk = n // _BIN
    ncol = n // _RB
    ncol_pad = -(-ncol // 128) * 128
    npad = ncol_pad * _RB

    x1 = box[:, 0]
    y1 = box[:, 1]
    x2 = box[:, 2]
    y2 = box[:, 3]
    area = (x2 - x1) * (y2 - y1)

    s = score[0, 0]
    idx = jnp.arange(n, dtype=jnp.int32)
    order = jnp.argsort(-s, stable=True)
    srank = jnp.zeros((n,), jnp.int32).at[order].set(idx)
    u = jnp.where(s > 0, srank.astype(jnp.float32),
                  _BIG + idx.astype(jnp.float32))[None, :]
    bigi = (_BIG + idx.astype(jnp.float32))[None, :]

    def rowt(a, val):
        ap = jnp.pad(a, ((0, 0), (0, npad - n)), constant_values=val)
        return ap.reshape(ncol_pad, _RB).T

    row_args = [rowt(x1, 0.), rowt(y1, 0.), rowt(x2, 0.), rowt(y2, 0.),
                rowt(area, 0.), rowt(u, _PAD), rowt(bigi, _PAD)]

    ref_l = jnp.stack([a[:, :_BIN]
                       for a in (x1, y1, x2, y2, area, u, bigi)])

    col3 = [a.reshape(nblk, 1, _BIN) for a in (x1, y1, x2, y2, area)]
    featb = feat[0].reshape(c, nblk, _BIN).transpose(1, 0, 2)

    col_spec = pl.BlockSpec((1, 1, _BIN), lambda j: (j, 0, 0))
    full2 = lambda a: pl.BlockSpec(a.shape, lambda j: (0,) * a.ndim)

    w_arrays = [Wm, bm[..., None], Wmo, bmo[:, None],
                Wp, bp[..., None], Wpo, bpo[:, None],
                Wg, bg[..., None], Wgo, bgo[:, None]]
    in_arrays = (col3 + [ref_l] + row_args + [featb] + w_arrays)
    in_specs = ([col_spec] * 5
                + [full2(ref_l)]
                + [full2(a) for a in row_args]
                + [pl.BlockSpec((1, c, _BIN), lambda j: (j, 0, 0))]
                + [full2(a) for a in w_arrays])

    out_shape = (
        jax.ShapeDtypeStruct((nblk, 1, _BIN), jnp.float32),
        jax.ShapeDtypeStruct((nblk, 4, _BIN), jnp.float32),
        jax.ShapeDtypeStruct((nblk, 4, _BIN), jnp.float32),
        jax.ShapeDtypeStruct((nblk, 1, _BIN), jnp.float32),
    )
    blk_out = lambda r: pl.BlockSpec((1, r, _BIN), lambda j: (j, 0, 0))
    out_specs = (blk_out(1), blk_out(4), blk_out(4), blk_out(1))

    import functools
    pi, mu, gamma, loc = pl.pallas_call(
        functools.partial(_kernel, ncol_pad // 128),
        grid=(nblk,),
        in_specs=in_specs,
        out_specs=out_specs,
        out_shape=out_shape,
        compiler_params=pltpu.CompilerParams(
            dimension_semantics=("arbitrary",)),
    )(*in_arrays)

    pi = pi.reshape(1, 1, n)
    mu = mu.transpose(1, 0, 2).reshape(1, 4, n)
    gamma = gamma.transpose(1, 0, 2).reshape(1, 4, n)
    loc = loc.reshape(1, 1, n)
    return (pi, mu, gamma, loc)


# speedup vs baseline: 3.2293x; 1.0005x over previous
import functools

import jax
import jax.numpy as jnp
from jax.experimental import pallas as pl
from jax.experimental.pallas import tpu as pltpu

_BIN = 1000
_RB = 8
_THRS = (0.4, 0.6, 0.8)
_GAMMA_F = 0.05
_RESCALE = 0.02
_EPS = 1e-12
_BIG = float(1 << 20)
_PAD = float(1 << 24)


def _lrelu(x):
    return jnp.where(x >= 0, x, 0.2 * x)


def _dot(a, b):
    return jax.lax.dot_general(
        a, b, (((1,), (0,)), ((), ())),
        precision=jax.lax.Precision.HIGHEST,
        preferred_element_type=jnp.float32)


def _iou(x1a, y1a, x2a, y2a, aa, x1b, y1b, x2b, y2b, ab):
    wx = jnp.maximum(jnp.minimum(x2a, x2b) - jnp.maximum(x1a, x1b), 0.0)
    wy = jnp.maximum(jnp.minimum(y2a, y2b) - jnp.maximum(y1a, y1b), 0.0)
    inter = wx * wy
    return inter / ((aa + ab) - inter)


def _kernel(nrows, x1c_ref, y1c_ref, x2c_ref, y2c_ref, ac_ref, fl_ref,
            x1t_ref, y1t_ref, x2t_ref, y2t_ref, at_ref, ut_ref, bt_ref,
            feat_ref, Wm_ref, bm_ref, Wmo_ref, bmo_ref,
            Wp_ref, bp_ref, Wpo_ref, bpo_ref,
            Wg_ref, bg_ref, Wgo_ref, bgo_ref,
            pi_ref, mu_ref, gamma_ref, loc_ref):
    x1c = x1c_ref[0]
    y1c = y1c_ref[0]
    x2c = x2c_ref[0]
    y2c = y2c_ref[0]
    ac = ac_ref[0]

    iou_rg = _iou(fl_ref[0], fl_ref[1], fl_ref[2], fl_ref[3], fl_ref[4],
                  x1c, y1c, x2c, y2c, ac)
    K = [jnp.where(iou_rg >= t, fl_ref[5], fl_ref[6]) for t in _THRS]

    row_refs = (x1t_ref, y1t_ref, x2t_ref, y2t_ref, at_ref, ut_ref, bt_ref)

    def rowstep(k, bad):
        base = pl.multiple_of(k * 128, 128)
        tiles = [r[:, pl.ds(base, 128)] for r in row_refs]
        out = bad
        for j in range(128):
            sj = slice(j, j + 1)
            x1i, y1i, x2i, y2i, ai, ui, bi = [t[:, sj] for t in tiles]
            iou = _iou(x1i, y1i, x2i, y2i, ai, x1c, y1c, x2c, y2c, ac)
            nxt = []
            for t, b in zip(_THRS, out):
                key = jnp.where(iou >= t, ui, bi)
                nxt.append(jnp.minimum(b, key))
            out = tuple(nxt)
        return out

    nrowsteps = nrows
    z = jnp.full((_RB, _BIN), _PAD, dtype=jnp.float32)
    minkey = jax.lax.fori_loop(0, nrowsteps, rowstep, (z, z, z))
    masks = [jnp.where(jnp.min(mk, axis=0, keepdims=True) < k_t, 0.0, 1.0)
             for mk, k_t in zip(minkey, K)]

    feat = feat_ref[0]

    def stack(h, Ws_ref, bs_ref, Wo_ref, bo_ref):
        for i in range(Ws_ref.shape[0]):
            h = _lrelu(_dot(Ws_ref[i], h) + bs_ref[i])
        return _dot(Wo_ref[...], h) + bo_ref[...]

    om = stack(feat, Wm_ref, bm_ref, Wmo_ref, bmo_ref)
    mx = jnp.max(om, axis=0, keepdims=True)
    e = jnp.exp(om - mx)
    w = e / jnp.sum(e, axis=0, keepdims=True)
    loc_ref[0] = (masks[0] * w[0:1] + masks[1] * w[1:2] + masks[2] * w[2:3])

    op = stack(feat, Wp_ref, bp_ref, Wpo_ref, bpo_ref)
    pi_ref[0] = jnp.exp(op)

    og = stack(feat, Wg_ref, bg_ref, Wgo_ref, bgo_ref)
    sp = jnp.maximum(og, 0.0) + jnp.log1p(jnp.exp(-jnp.abs(og)))
    mu1 = _RESCALE * x1c
    mu2 = _RESCALE * y1c
    mu3 = _RESCALE * x2c
    mu4 = _RESCALE * y2c
    dx = jnp.maximum(mu3 - mu1, _EPS)
    dy = jnp.maximum(mu4 - mu2, _EPS)
    ming = _GAMMA_F * jnp.concatenate([dx, dy, dx, dy], axis=0)
    gamma_ref[0] = sp + ming
    mu_ref[0] = jnp.concatenate([mu1, mu2, mu3, mu4], axis=0)


def kernel(box, score, feat, Wm, bm, Wmo, bmo, Wp, bp, Wpo, bpo, Wg, bg, Wgo, bgo):
    n = box.shape[2]
    c = feat.shape[1]
    nblk = n // _BIN
    ncol = n // _RB
    ncol_pad = -(-ncol // 128) * 128
    npad = ncol_pad * _RB

    x1 = box[:, 0]
    y1 = box[:, 1]
    x2 = box[:, 2]
    y2 = box[:, 3]
    area = (x2 - x1) * (y2 - y1)

    s = score[0, 0]
    idx = jnp.arange(n, dtype=jnp.int32)
    order = jnp.argsort(-s, stable=True)
    srank = jnp.zeros((n,), jnp.int32).at[order].set(idx)
    u = jnp.where(s > 0, srank.astype(jnp.float32),
                  _BIG + idx.astype(jnp.float32))[None, :]
    bigi = (_BIG + idx.astype(jnp.float32))[None, :]

    def rowt(a, val):
        ap = jnp.pad(a, ((0, 0), (0, npad - n)), constant_values=val)
        return ap.reshape(ncol_pad, _RB).T

    row_args = [rowt(x1, 0.), rowt(y1, 0.), rowt(x2, 0.), rowt(y2, 0.),
                rowt(area, 0.), rowt(u, _PAD), rowt(bigi, _PAD)]

    ref_l = jnp.stack([a[:, :_BIN]
                       for a in (x1, y1, x2, y2, area, u, bigi)])

    col3 = [a.reshape(nblk, 1, _BIN) for a in (x1, y1, x2, y2, area)]
    featb = feat[0].reshape(c, nblk, _BIN).transpose(1, 0, 2)

    col_spec = pl.BlockSpec((1, 1, _BIN), lambda j: (j, 0, 0))
    full2 = lambda a: pl.BlockSpec(a.shape, lambda j: (0,) * a.ndim)

    w_arrays = [Wm, bm[..., None], Wmo, bmo[:, None],
                Wp, bp[..., None], Wpo, bpo[:, None],
                Wg, bg[..., None], Wgo, bgo[:, None]]
    in_arrays = (col3 + [ref_l] + row_args + [featb] + w_arrays)
    in_specs = ([col_spec] * 5
                + [full2(ref_l)]
                + [full2(a) for a in row_args]
                + [pl.BlockSpec((1, c, _BIN), lambda j: (j, 0, 0))]
                + [full2(a) for a in w_arrays])

    out_shape = (
        jax.ShapeDtypeStruct((nblk, 1, _BIN), jnp.float32),
        jax.ShapeDtypeStruct((nblk, 4, _BIN), jnp.float32),
        jax.ShapeDtypeStruct((nblk, 4, _BIN), jnp.float32),
        jax.ShapeDtypeStruct((nblk, 1, _BIN), jnp.float32),
    )
    blk_out = lambda r: pl.BlockSpec((1, r, _BIN), lambda j: (j, 0, 0))
    out_specs = (blk_out(1), blk_out(4), blk_out(4), blk_out(1))

    pi, mu, gamma, loc = pl.pallas_call(
        functools.partial(_kernel, ncol_pad // 128),
        grid=(nblk,),
        in_specs=in_specs,
        out_specs=out_specs,
        out_shape=out_shape,
        compiler_params=pltpu.CompilerParams(
            dimension_semantics=("arbitrary",)),
    )(*in_arrays)

    pi = pi.reshape(1, 1, n)
    mu = mu.transpose(1, 0, 2).reshape(1, 4, n)
    gamma = gamma.transpose(1, 0, 2).reshape(1, 4, n)
    loc = loc.reshape(1, 1, n)
    return (pi, mu, gamma, loc)
